# Initial kernel scaffold; baseline (speedup 1.0000x reference)
import jax
import jax.numpy as jnp
from jax.experimental import pallas as pl


def kernel(input_ids, token_type_ids, token_table, seg_table, W, b, rms_weight):
    raise NotImplementedError("write your pallas kernel here")


# speedup vs baseline: 1.4582x; 1.4582x over previous
import jax
import jax.numpy as jnp
from jax.experimental import pallas as pl
from jax.experimental.pallas import tpu as pltpu
from jax.experimental.pallas import tpu_sc as plsc

_EMB = 128
_HID = 768
_GW = 128
_TB = 1024


def _sc_gather(token_table, ids_flat):
    n = ids_flat.shape[0]
    ids2 = ids_flat.reshape(1, n)
    mesh = plsc.VectorSubcoreMesh(core_axis_name="core",
                                  subcore_axis_name="subcore")

    @pl.kernel(out_type=jax.ShapeDtypeStruct((n, _EMB), token_table.dtype),
               mesh=mesh)
    def gk(tbl_hbm, i_hbm, o_hbm):
        def body(i_vmem, o_vmem):
            pltpu.sync_copy(tbl_hbm.at[i_vmem.at[0]], o_vmem)

        pltpu.emit_pipeline(
            body,
            grid=(n // _GW,),
            in_specs=[pl.BlockSpec((1, _GW), lambda i: (0, i))],
            out_specs=[pl.BlockSpec((_GW, _EMB), lambda i: (i, 0))],
            core_axis_name=("core", "subcore"),
            dimension_semantics=(pltpu.PARALLEL,),
        )(i_hbm, o_hbm)

    return gk(token_table, ids2)


def _tc_body(g_ref, ttf_ref, seg_ref, wt_ref, b_ref, rw_ref, o_ref):
    seg0 = seg_ref[0:1, :]
    dseg = seg_ref[1:2, :] - seg0
    x = g_ref[...] + seg0 + ttf_ref[...] * dseg
    y = jax.lax.dot_general(
        x, wt_ref[...], (((1,), (0,)), ((), ())),
        preferred_element_type=jnp.float32,
        precision=jax.lax.Precision.HIGHEST,
    ) + b_ref[...]
    var = jnp.mean(y * y, axis=-1, keepdims=True)
    o_ref[...] = y * jax.lax.rsqrt(var + 1e-6) * rw_ref[...]


def _tc_project(g, ttf, seg_table, wt, b2, rw2):
    n = g.shape[0]
    return pl.pallas_call(
        _tc_body,
        grid=(n // _TB,),
        in_specs=[
            pl.BlockSpec((_TB, _EMB), lambda i: (i, 0)),
            pl.BlockSpec((_TB, 1), lambda i: (i, 0)),
            pl.BlockSpec((2, _EMB), lambda i: (0, 0)),
            pl.BlockSpec((_EMB, _HID), lambda i: (0, 0)),
            pl.BlockSpec((1, _HID), lambda i: (0, 0)),
            pl.BlockSpec((1, _HID), lambda i: (0, 0)),
        ],
        out_specs=pl.BlockSpec((_TB, _HID), lambda i: (i, 0)),
        out_shape=jax.ShapeDtypeStruct((n, _HID), jnp.float32),
    )(g, ttf, seg_table, wt, b2, rw2)


def kernel(input_ids, token_type_ids, token_table, seg_table, W, b, rms_weight):
    bsz, seq = input_ids.shape
    n = bsz * seq
    g = _sc_gather(token_table, input_ids.reshape(n))
    ttf = token_type_ids.reshape(n, 1).astype(jnp.float32)
    out = _tc_project(g, ttf, seg_table, W.T,
                      b.reshape(1, _HID), rms_weight.reshape(1, _HID))
    return out.reshape(bsz, seq, _HID)


# speedup vs baseline: 1.8778x; 1.2877x over previous
import jax
import jax.numpy as jnp
from jax.experimental import pallas as pl
from jax.experimental.pallas import tpu as pltpu
from jax.experimental.pallas import tpu_sc as plsc

_EMB = 128
_HID = 768
_GW = 128
_TB = 1024


def _sc_gather(token_table, ids_flat):
    n = ids_flat.shape[0]
    ids2 = ids_flat.reshape(1, n)
    mesh = plsc.VectorSubcoreMesh(core_axis_name="core",
                                  subcore_axis_name="subcore")

    @pl.kernel(out_type=jax.ShapeDtypeStruct((n, _EMB), token_table.dtype),
               mesh=mesh)
    def gk(tbl_hbm, i_hbm, o_hbm):
        def body(i_vmem, o_vmem):
            pltpu.sync_copy(tbl_hbm.at[i_vmem.at[0]], o_vmem)

        pltpu.emit_pipeline(
            body,
            grid=(n // _GW,),
            in_specs=[pl.BlockSpec((1, _GW), lambda i: (0, i))],
            out_specs=[pl.BlockSpec((_GW, _EMB), lambda i: (i, 0))],
            core_axis_name=("core", "subcore"),
            dimension_semantics=(pltpu.PARALLEL,),
        )(i_hbm, o_hbm)

    return gk(token_table, ids2)


def _tc_body(g_ref, ttf_ref, seg_ref, wt_ref, b_ref, rw_ref, o_ref):
    seg0 = seg_ref[0:1, :]
    dseg = seg_ref[1:2, :] - seg0
    x = g_ref[...] + seg0 + ttf_ref[...] * dseg
    y = jax.lax.dot_general(
        x, wt_ref[...], (((1,), (0,)), ((), ())),
        preferred_element_type=jnp.float32,
        precision=jax.lax.Precision.DEFAULT,
    ) + b_ref[...]
    var = jnp.mean(y * y, axis=-1, keepdims=True)
    o_ref[...] = y * jax.lax.rsqrt(var + 1e-6) * rw_ref[...]


def _tc_project(g, ttf, seg_table, wt, b2, rw2):
    n = g.shape[0]
    return pl.pallas_call(
        _tc_body,
        grid=(n // _TB,),
        in_specs=[
            pl.BlockSpec((_TB, _EMB), lambda i: (i, 0)),
            pl.BlockSpec((_TB, 1), lambda i: (i, 0)),
            pl.BlockSpec((2, _EMB), lambda i: (0, 0)),
            pl.BlockSpec((_EMB, _HID), lambda i: (0, 0)),
            pl.BlockSpec((1, _HID), lambda i: (0, 0)),
            pl.BlockSpec((1, _HID), lambda i: (0, 0)),
        ],
        out_specs=pl.BlockSpec((_TB, _HID), lambda i: (i, 0)),
        out_shape=jax.ShapeDtypeStruct((n, _HID), jnp.float32),
    )(g, ttf, seg_table, wt, b2, rw2)


def kernel(input_ids, token_type_ids, token_table, seg_table, W, b, rms_weight):
    bsz, seq = input_ids.shape
    n = bsz * seq
    g = _sc_gather(token_table, input_ids.reshape(n))
    ttf = token_type_ids.reshape(n, 1).astype(jnp.float32)
    out = _tc_project(g, ttf, seg_table, W.T,
                      b.reshape(1, _HID), rms_weight.reshape(1, _HID))
    return out.reshape(bsz, seq, _HID)
